# Initial kernel scaffold; baseline (speedup 1.0000x reference)
import functools

import jax
import jax.numpy as jnp
from jax import lax
from jax.experimental import pallas as pl
from jax.experimental.pallas import tpu as pltpu
from jax.experimental.pallas import tpu_sc as plsc

N = 10000
E = 320000
NPAD = 10240
NC, NS = 2, 16
NW = NC * NS
EPAD = 327680
EPT = EPAD // NW
K = 128
RPT = NPAD // NS
R = 1024
GRID = NPAD // R

_mesh = plsc.VectorSubcoreMesh(core_axis_name="c", subcore_axis_name="s")


def _deg_body(src_hbm, dst_hbm, ones_hbm, zeros_hbm, out_hbm,
              sidx, didx, ones_v, acc_o, acc_i, sem):
    cid = lax.axis_index("c")
    sid = lax.axis_index("s")
    wid = cid * NS + sid
    pltpu.sync_copy(ones_hbm, ones_v)
    pltpu.sync_copy(zeros_hbm, acc_o.at[pl.ds(sid * RPT, RPT)])
    pltpu.sync_copy(zeros_hbm, acc_i.at[pl.ds(sid * RPT, RPT)])
    plsc.subcore_barrier()
    base = wid * EPT

    def blk(i, carry):
        off = base + i * K
        pltpu.sync_copy(src_hbm.at[pl.ds(off, K)], sidx)
        pltpu.sync_copy(dst_hbm.at[pl.ds(off, K)], didx)
        pltpu.sync_copy(ones_v, acc_o.at[sidx], add=True)
        pltpu.sync_copy(ones_v, acc_i.at[didx], add=True)
        return carry

    lax.fori_loop(0, EPT // K, blk, 0)
    plsc.subcore_barrier()
    sl = pl.ds(sid * RPT, RPT)
    pltpu.sync_copy(acc_o.at[sl], out_hbm.at[cid, 0, sl])
    pltpu.sync_copy(acc_i.at[sl], out_hbm.at[cid, 1, sl])


_deg_call = pl.kernel(
    _deg_body,
    out_type=jax.ShapeDtypeStruct((NC, 2, NPAD, 16), jnp.float32),
    mesh=_mesh,
    scratch_types=[
        pltpu.VMEM((K,), jnp.int32),
        pltpu.VMEM((K,), jnp.int32),
        pltpu.VMEM((K, 16), jnp.float32),
        pltpu.VMEM_SHARED((NPAD, 16), jnp.float32),
        pltpu.VMEM_SHARED((NPAD, 16), jnp.float32),
        pltpu.SemaphoreType.DMA,
    ],
)


def _make_agg(C):
    def body(*refs):
        tables = refs[:C]
        src_hbm, dst_hbm, zeros_hbm, out_hbm = refs[C:C + 4]
        sidx, didx, rows, acc, sem = refs[C + 4:]
        cid = lax.axis_index("c")
        sid = lax.axis_index("s")
        wid = cid * NS + sid
        base = wid * EPT
        sl = pl.ds(sid * RPT, RPT)
        for c in range(C):
            pltpu.sync_copy(zeros_hbm, acc.at[sl])
            plsc.subcore_barrier()
            table = tables[c]

            def blk(i, carry):
                off = base + i * K
                pltpu.sync_copy(src_hbm.at[pl.ds(off, K)], sidx)
                pltpu.sync_copy(dst_hbm.at[pl.ds(off, K)], didx)
                pltpu.async_copy(table.at[sidx], rows, sem).wait()
                pltpu.sync_copy(rows, acc.at[didx], add=True)
                return carry

            lax.fori_loop(0, EPT // K, blk, 0)
            plsc.subcore_barrier()
            pltpu.sync_copy(acc.at[sl], out_hbm.at[cid, c, sl])
            if c != C - 1:
                plsc.subcore_barrier()

    return pl.kernel(
        body,
        out_type=jax.ShapeDtypeStruct((NC, C, NPAD, 128), jnp.float32),
        mesh=_mesh,
        scratch_types=[
            pltpu.VMEM((K,), jnp.int32),
            pltpu.VMEM((K,), jnp.int32),
            pltpu.VMEM((K, 128), jnp.float32),
            pltpu.VMEM_SHARED((NPAD, 128), jnp.float32),
            pltpu.SemaphoreType.DMA,
        ],
    )


_agg1 = _make_agg(1)
_agg4 = _make_agg(4)
_agg2 = _make_agg(2)


def _prep_body(deg_ref, x_ref, nout_ref, nin_ref, c1_ref):
    dego = deg_ref[0, 0] + deg_ref[1, 0]
    degi = deg_ref[0, 1] + deg_ref[1, 1]
    no = lax.rsqrt(jnp.maximum(dego, 1.0))
    ni = lax.rsqrt(jnp.maximum(degi, 1.0))
    nout_ref[...] = no
    nin_ref[...] = ni
    c1_ref[...] = x_ref[...] * no[:, :1]


_prep_call = pl.pallas_call(
    _prep_body,
    grid=(GRID,),
    in_specs=[
        pl.BlockSpec((NC, 2, R, 16), lambda i: (0, 0, i, 0)),
        pl.BlockSpec((R, 128), lambda i: (i, 0)),
    ],
    out_specs=[
        pl.BlockSpec((R, 16), lambda i: (i, 0)),
        pl.BlockSpec((R, 16), lambda i: (i, 0)),
        pl.BlockSpec((R, 128), lambda i: (i, 0)),
    ],
    out_shape=[
        jax.ShapeDtypeStruct((NPAD, 16), jnp.float32),
        jax.ShapeDtypeStruct((NPAD, 16), jnp.float32),
        jax.ShapeDtypeStruct((NPAD, 128), jnp.float32),
    ],
)


def _layer1_body(r_ref, nin_ref, nout_ref, w_ref, out_ref):
    agg = (r_ref[0, 0] + r_ref[1, 0]) * nin_ref[:, :1]
    z = jnp.dot(agg, w_ref[...], preferred_element_type=jnp.float32)
    t = jnp.maximum(z, 0.0) * nout_ref[:, :1]
    for c in range(4):
        out_ref[c] = t[:, c * 128:(c + 1) * 128]


_layer1_call = pl.pallas_call(
    _layer1_body,
    grid=(GRID,),
    in_specs=[
        pl.BlockSpec((NC, 1, R, 128), lambda i: (0, 0, i, 0)),
        pl.BlockSpec((R, 16), lambda i: (i, 0)),
        pl.BlockSpec((R, 16), lambda i: (i, 0)),
        pl.BlockSpec((128, 512), lambda i: (0, 0)),
    ],
    out_specs=pl.BlockSpec((4, R, 128), lambda i: (0, i, 0)),
    out_shape=jax.ShapeDtypeStruct((4, NPAD, 128), jnp.float32),
)


def _layer23_body(r_ref, nin_ref, nout_ref, w2_ref, w3_ref, out_ref):
    h = jnp.concatenate(
        [r_ref[0, c] + r_ref[1, c] for c in range(4)], axis=1
    ) * nin_ref[:, :1]
    z = jnp.maximum(jnp.dot(h, w2_ref[...], preferred_element_type=jnp.float32), 0.0)
    g = jnp.dot(z, w3_ref[...], preferred_element_type=jnp.float32)
    t = g * nout_ref[:, :1]
    for c in range(2):
        out_ref[c] = t[:, c * 128:(c + 1) * 128]


_layer23_call = pl.pallas_call(
    _layer23_body,
    grid=(GRID,),
    in_specs=[
        pl.BlockSpec((NC, 4, R, 128), lambda i: (0, 0, i, 0)),
        pl.BlockSpec((R, 16), lambda i: (i, 0)),
        pl.BlockSpec((R, 16), lambda i: (i, 0)),
        pl.BlockSpec((512, 512), lambda i: (0, 0)),
        pl.BlockSpec((512, 256), lambda i: (0, 0)),
    ],
    out_specs=pl.BlockSpec((2, R, 128), lambda i: (0, i, 0)),
    out_shape=jax.ShapeDtypeStruct((2, NPAD, 128), jnp.float32),
)


def _final_body(r_ref, nin_ref, out_ref):
    out_ref[...] = jnp.concatenate(
        [r_ref[0, c] + r_ref[1, c] for c in range(2)], axis=1
    ) * nin_ref[:, :1]


_final_call = pl.pallas_call(
    _final_body,
    grid=(GRID,),
    in_specs=[
        pl.BlockSpec((NC, 2, R, 128), lambda i: (0, 0, i, 0)),
        pl.BlockSpec((R, 16), lambda i: (i, 0)),
    ],
    out_specs=pl.BlockSpec((R, 256), lambda i: (i, 0)),
    out_shape=jax.ShapeDtypeStruct((NPAD, 256), jnp.float32),
)


@jax.jit
def kernel(x, edge_index, W1, W2, W3):
    src = edge_index[0].astype(jnp.int32)
    dst = edge_index[1].astype(jnp.int32)
    pad = jnp.full((EPAD - E,), N, jnp.int32)
    src_p = jnp.concatenate([src, pad])
    dst_p = jnp.concatenate([dst, pad])
    x_pad = jnp.pad(x, ((0, NPAD - N), (0, 0)))
    ones16 = jnp.ones((K, 16), jnp.float32)
    zeros128 = jnp.zeros((RPT, 128), jnp.float32)
    zeros16 = jnp.zeros((RPT, 16), jnp.float32)

    deg = _deg_call(src_p, dst_p, ones16, zeros16)
    nout16, nin16, c1 = _prep_call(deg, x_pad)
    r1 = _agg1(c1, src_p, dst_p, zeros128)
    c2 = _layer1_call(r1, nin16, nout16, W1)
    r2 = _agg4(c2[0], c2[1], c2[2], c2[3], src_p, dst_p, zeros128)
    c3 = _layer23_call(r2, nin16, nout16, W2, W3)
    r3 = _agg2(c3[0], c3[1], src_p, dst_p, zeros128)
    out = _final_call(r3, nin16)
    return out[:N]


# speedup vs baseline: 2.8565x; 2.8565x over previous
import functools

import jax
import jax.numpy as jnp
from jax import lax
from jax.experimental import pallas as pl
from jax.experimental.pallas import tpu as pltpu
from jax.experimental.pallas import tpu_sc as plsc

N = 10000
E = 320000
NPAD = 10240
NC, NS = 2, 16
NW = NC * NS
EPAD = 327680
EPT = EPAD // NW
K = 128
RPT = NPAD // NS
R = 1024
GRID = NPAD // R

_mesh = plsc.VectorSubcoreMesh(core_axis_name="c", subcore_axis_name="s")


def _deg_body(src_hbm, dst_hbm, ones_hbm, zeros_hbm, out_hbm,
              idx, ones_v, acc, sem):
    cid = lax.axis_index("c")
    sid = lax.axis_index("s")
    wid = cid * NS + sid
    base = wid * EPT
    sl = pl.ds(sid * RPT, RPT)
    pltpu.sync_copy(ones_hbm, ones_v)
    for phase in range(2):
        idx_hbm = (src_hbm, dst_hbm)[phase]
        pltpu.sync_copy(zeros_hbm, acc.at[sl])
        plsc.subcore_barrier()

        def blk(i, carry):
            off = base + i * K
            pltpu.sync_copy(idx_hbm.at[pl.ds(off, K)], idx)
            pltpu.sync_copy(ones_v, acc.at[idx], add=True)
            return carry

        lax.fori_loop(0, EPT // K, blk, 0)
        plsc.subcore_barrier()
        pltpu.sync_copy(acc.at[sl], out_hbm.at[cid, phase, sl])
        if phase == 0:
            plsc.subcore_barrier()


_deg_call = pl.kernel(
    _deg_body,
    out_type=jax.ShapeDtypeStruct((NC, 2, NPAD, 128), jnp.float32),
    mesh=_mesh,
    scratch_types=[
        pltpu.VMEM((K,), jnp.int32),
        pltpu.VMEM((K, 128), jnp.float32),
        pltpu.VMEM_SHARED((NPAD, 128), jnp.float32),
        pltpu.SemaphoreType.DMA,
    ],
)


def _make_agg(C):
    def body(*refs):
        tables = refs[:C]
        src_hbm, dst_hbm, zeros_hbm, out_hbm = refs[C:C + 4]
        sidx, didx, rows, acc, sem = refs[C + 4:]
        cid = lax.axis_index("c")
        sid = lax.axis_index("s")
        wid = cid * NS + sid
        base = wid * EPT
        sl = pl.ds(sid * RPT, RPT)
        for c in range(C):
            pltpu.sync_copy(zeros_hbm, acc.at[sl])
            plsc.subcore_barrier()
            table = tables[c]

            def blk(i, carry):
                off = base + i * K
                pltpu.sync_copy(src_hbm.at[pl.ds(off, K)], sidx)
                pltpu.sync_copy(dst_hbm.at[pl.ds(off, K)], didx)
                pltpu.async_copy(table.at[sidx], rows, sem).wait()
                pltpu.sync_copy(rows, acc.at[didx], add=True)
                return carry

            lax.fori_loop(0, EPT // K, blk, 0)
            plsc.subcore_barrier()
            pltpu.sync_copy(acc.at[sl], out_hbm.at[cid, c, sl])
            if c != C - 1:
                plsc.subcore_barrier()

    return pl.kernel(
        body,
        out_type=jax.ShapeDtypeStruct((NC, C, NPAD, 128), jnp.float32),
        mesh=_mesh,
        scratch_types=[
            pltpu.VMEM((K,), jnp.int32),
            pltpu.VMEM((K,), jnp.int32),
            pltpu.VMEM((K, 128), jnp.float32),
            pltpu.VMEM_SHARED((NPAD, 128), jnp.float32),
            pltpu.SemaphoreType.DMA,
        ],
    )


_agg1 = _make_agg(1)
_agg4 = _make_agg(4)
_agg2 = _make_agg(2)


def _prep_body(deg_ref, x_ref, nout_ref, nin_ref, c1_ref):
    dego = deg_ref[0, 0] + deg_ref[1, 0]
    degi = deg_ref[0, 1] + deg_ref[1, 1]
    no = lax.rsqrt(jnp.maximum(dego, 1.0))
    ni = lax.rsqrt(jnp.maximum(degi, 1.0))
    nout_ref[...] = no[:, :16]
    nin_ref[...] = ni[:, :16]
    c1_ref[...] = x_ref[...] * no[:, :1]


_prep_call = pl.pallas_call(
    _prep_body,
    grid=(GRID,),
    in_specs=[
        pl.BlockSpec((NC, 2, R, 128), lambda i: (0, 0, i, 0)),
        pl.BlockSpec((R, 128), lambda i: (i, 0)),
    ],
    out_specs=[
        pl.BlockSpec((R, 16), lambda i: (i, 0)),
        pl.BlockSpec((R, 16), lambda i: (i, 0)),
        pl.BlockSpec((R, 128), lambda i: (i, 0)),
    ],
    out_shape=[
        jax.ShapeDtypeStruct((NPAD, 16), jnp.float32),
        jax.ShapeDtypeStruct((NPAD, 16), jnp.float32),
        jax.ShapeDtypeStruct((NPAD, 128), jnp.float32),
    ],
)


def _layer1_body(r_ref, nin_ref, nout_ref, w_ref, out_ref):
    agg = (r_ref[0, 0] + r_ref[1, 0]) * nin_ref[:, :1]
    z = jnp.dot(agg, w_ref[...], preferred_element_type=jnp.float32)
    t = jnp.maximum(z, 0.0) * nout_ref[:, :1]
    for c in range(4):
        out_ref[c] = t[:, c * 128:(c + 1) * 128]


_layer1_call = pl.pallas_call(
    _layer1_body,
    grid=(GRID,),
    in_specs=[
        pl.BlockSpec((NC, 1, R, 128), lambda i: (0, 0, i, 0)),
        pl.BlockSpec((R, 16), lambda i: (i, 0)),
        pl.BlockSpec((R, 16), lambda i: (i, 0)),
        pl.BlockSpec((128, 512), lambda i: (0, 0)),
    ],
    out_specs=pl.BlockSpec((4, R, 128), lambda i: (0, i, 0)),
    out_shape=jax.ShapeDtypeStruct((4, NPAD, 128), jnp.float32),
)


def _layer23_body(r_ref, nin_ref, nout_ref, w2_ref, w3_ref, out_ref):
    h = jnp.concatenate(
        [r_ref[0, c] + r_ref[1, c] for c in range(4)], axis=1
    ) * nin_ref[:, :1]
    z = jnp.maximum(jnp.dot(h, w2_ref[...], preferred_element_type=jnp.float32), 0.0)
    g = jnp.dot(z, w3_ref[...], preferred_element_type=jnp.float32)
    t = g * nout_ref[:, :1]
    for c in range(2):
        out_ref[c] = t[:, c * 128:(c + 1) * 128]


_layer23_call = pl.pallas_call(
    _layer23_body,
    grid=(GRID,),
    in_specs=[
        pl.BlockSpec((NC, 4, R, 128), lambda i: (0, 0, i, 0)),
        pl.BlockSpec((R, 16), lambda i: (i, 0)),
        pl.BlockSpec((R, 16), lambda i: (i, 0)),
        pl.BlockSpec((512, 512), lambda i: (0, 0)),
        pl.BlockSpec((512, 256), lambda i: (0, 0)),
    ],
    out_specs=pl.BlockSpec((2, R, 128), lambda i: (0, i, 0)),
    out_shape=jax.ShapeDtypeStruct((2, NPAD, 128), jnp.float32),
)


def _final_body(r_ref, nin_ref, out_ref):
    out_ref[...] = jnp.concatenate(
        [r_ref[0, c] + r_ref[1, c] for c in range(2)], axis=1
    ) * nin_ref[:, :1]


_final_call = pl.pallas_call(
    _final_body,
    grid=(GRID,),
    in_specs=[
        pl.BlockSpec((NC, 2, R, 128), lambda i: (0, 0, i, 0)),
        pl.BlockSpec((R, 16), lambda i: (i, 0)),
    ],
    out_specs=pl.BlockSpec((R, 256), lambda i: (i, 0)),
    out_shape=jax.ShapeDtypeStruct((NPAD, 256), jnp.float32),
)


@jax.jit
def kernel(x, edge_index, W1, W2, W3):
    src = edge_index[0].astype(jnp.int32)
    dst = edge_index[1].astype(jnp.int32)
    pad = jnp.full((EPAD - E,), N, jnp.int32)
    src_p = jnp.concatenate([src, pad])
    dst_p = jnp.concatenate([dst, pad])
    x_pad = jnp.pad(x, ((0, NPAD - N), (0, 0)))
    ones128 = jnp.ones((K, 128), jnp.float32)
    zeros128 = jnp.zeros((RPT, 128), jnp.float32)

    deg = _deg_call(src_p, dst_p, ones128, zeros128)
    nout16, nin16, c1 = _prep_call(deg, x_pad)
    r1 = _agg1(c1, src_p, dst_p, zeros128)
    c2 = _layer1_call(r1, nin16, nout16, W1)
    r2 = _agg4(c2[0], c2[1], c2[2], c2[3], src_p, dst_p, zeros128)
    c3 = _layer23_call(r2, nin16, nout16, W2, W3)
    r3 = _agg2(c3[0], c3[1], src_p, dst_p, zeros128)
    out = _final_call(r3, nin16)
    return out[:N]


# speedup vs baseline: 3.3321x; 1.1665x over previous
import functools

import jax
import jax.numpy as jnp
from jax import lax
from jax.experimental import pallas as pl
from jax.experimental.pallas import tpu as pltpu
from jax.experimental.pallas import tpu_sc as plsc

N = 10000
E = 320000
NPAD = 10240
NC, NS = 2, 16
NW = NC * NS
EPAD = 327680
EPT = EPAD // NW
K = 128
NBLK = EPT // K
HALF = NBLK // 2
DG = 8
RPT = NPAD // NS
R = 1024
GRID = NPAD // R

_mesh = plsc.VectorSubcoreMesh(core_axis_name="c", subcore_axis_name="s")


def _deg_body(src_hbm, dst_hbm, ones_hbm, zeros_hbm, out_hbm,
              idx, ones_v, acc, sem):
    cid = lax.axis_index("c")
    sid = lax.axis_index("s")
    wid = cid * NS + sid
    sl = pl.ds(sid * RPT, RPT)
    pltpu.sync_copy(ones_hbm, ones_v)
    for phase in range(2):
        idx_hbm = (src_hbm, dst_hbm)[phase]
        pltpu.sync_copy(idx_hbm.at[wid], idx)
        pltpu.sync_copy(zeros_hbm, acc.at[sl])
        plsc.subcore_barrier()

        def blk(j, carry):
            descs = [
                pltpu.async_copy(ones_v, acc.at[idx.at[j * DG + b]],
                                 sem, add=True)
                for b in range(DG)
            ]
            for d_ in descs:
                d_.wait()
            return carry

        lax.fori_loop(0, NBLK // DG, blk, 0)
        plsc.subcore_barrier()
        pltpu.sync_copy(acc.at[sl], out_hbm.at[cid, phase, sl])
        if phase == 0:
            plsc.subcore_barrier()


_deg_call = pl.kernel(
    _deg_body,
    out_type=jax.ShapeDtypeStruct((NC, 2, NPAD, 128), jnp.float32),
    mesh=_mesh,
    scratch_types=[
        pltpu.VMEM((NBLK, K), jnp.int32),
        pltpu.VMEM((K, 128), jnp.float32),
        pltpu.VMEM_SHARED((NPAD, 128), jnp.float32),
        pltpu.SemaphoreType.DMA,
    ],
)


def _make_agg(C):
    def body(*refs):
        tables = refs[:C]
        src_hbm, dst_hbm, zeros_hbm, out_hbm = refs[C:C + 4]
        sidx, didx, rows, acc, semg, sems = refs[C + 4:]
        cid = lax.axis_index("c")
        sid = lax.axis_index("s")
        wid = cid * NS + sid
        sl = pl.ds(sid * RPT, RPT)
        for c in range(C):
            pltpu.sync_copy(zeros_hbm, acc.at[sl])
            plsc.subcore_barrier()
            table = tables[c]
            for h in range(2):
                pltpu.sync_copy(src_hbm.at[wid, pl.ds(h * HALF, HALF)], sidx)
                pltpu.sync_copy(dst_hbm.at[wid, pl.ds(h * HALF, HALF)], didx)

                def grp(j, carry):
                    gA = pltpu.async_copy(table.at[sidx.at[2 * j]],
                                          rows.at[0], semg)
                    gA.wait()
                    gB = pltpu.async_copy(table.at[sidx.at[2 * j + 1]],
                                          rows.at[1], semg)
                    sA = pltpu.async_copy(rows.at[0],
                                          acc.at[didx.at[2 * j]],
                                          sems, add=True)
                    sA.wait()
                    gB.wait()
                    sB = pltpu.async_copy(rows.at[1],
                                          acc.at[didx.at[2 * j + 1]],
                                          sems, add=True)
                    sB.wait()
                    return carry

                lax.fori_loop(0, HALF // 2, grp, 0)
            plsc.subcore_barrier()
            pltpu.sync_copy(acc.at[sl], out_hbm.at[cid, c, sl])
            if c != C - 1:
                plsc.subcore_barrier()

    return pl.kernel(
        body,
        out_type=jax.ShapeDtypeStruct((NC, C, NPAD, 128), jnp.float32),
        mesh=_mesh,
        scratch_types=[
            pltpu.VMEM((HALF, K), jnp.int32),
            pltpu.VMEM((HALF, K), jnp.int32),
            pltpu.VMEM((2, K, 128), jnp.float32),
            pltpu.VMEM_SHARED((NPAD, 128), jnp.float32),
            pltpu.SemaphoreType.DMA,
            pltpu.SemaphoreType.DMA,
        ],
    )


_agg1 = _make_agg(1)
_agg4 = _make_agg(4)
_agg2 = _make_agg(2)


def _prep_body(deg_ref, x_ref, nout_ref, nin_ref, c1_ref):
    dego = deg_ref[0, 0] + deg_ref[1, 0]
    degi = deg_ref[0, 1] + deg_ref[1, 1]
    no = lax.rsqrt(jnp.maximum(dego, 1.0))
    ni = lax.rsqrt(jnp.maximum(degi, 1.0))
    nout_ref[...] = no[:, :16]
    nin_ref[...] = ni[:, :16]
    c1_ref[...] = x_ref[...] * no[:, :1]


_prep_call = pl.pallas_call(
    _prep_body,
    grid=(GRID,),
    in_specs=[
        pl.BlockSpec((NC, 2, R, 128), lambda i: (0, 0, i, 0)),
        pl.BlockSpec((R, 128), lambda i: (i, 0)),
    ],
    out_specs=[
        pl.BlockSpec((R, 16), lambda i: (i, 0)),
        pl.BlockSpec((R, 16), lambda i: (i, 0)),
        pl.BlockSpec((R, 128), lambda i: (i, 0)),
    ],
    out_shape=[
        jax.ShapeDtypeStruct((NPAD, 16), jnp.float32),
        jax.ShapeDtypeStruct((NPAD, 16), jnp.float32),
        jax.ShapeDtypeStruct((NPAD, 128), jnp.float32),
    ],
)


def _layer1_body(r_ref, nin_ref, nout_ref, w_ref, out_ref):
    agg = (r_ref[0, 0] + r_ref[1, 0]) * nin_ref[:, :1]
    z = jnp.dot(agg, w_ref[...], preferred_element_type=jnp.float32)
    t = jnp.maximum(z, 0.0) * nout_ref[:, :1]
    for c in range(4):
        out_ref[c] = t[:, c * 128:(c + 1) * 128]


_layer1_call = pl.pallas_call(
    _layer1_body,
    grid=(GRID,),
    in_specs=[
        pl.BlockSpec((NC, 1, R, 128), lambda i: (0, 0, i, 0)),
        pl.BlockSpec((R, 16), lambda i: (i, 0)),
        pl.BlockSpec((R, 16), lambda i: (i, 0)),
        pl.BlockSpec((128, 512), lambda i: (0, 0)),
    ],
    out_specs=pl.BlockSpec((4, R, 128), lambda i: (0, i, 0)),
    out_shape=jax.ShapeDtypeStruct((4, NPAD, 128), jnp.float32),
)


def _layer23_body(r_ref, nin_ref, nout_ref, w2_ref, w3_ref, out_ref):
    h = jnp.concatenate(
        [r_ref[0, c] + r_ref[1, c] for c in range(4)], axis=1
    ) * nin_ref[:, :1]
    z = jnp.maximum(jnp.dot(h, w2_ref[...], preferred_element_type=jnp.float32), 0.0)
    g = jnp.dot(z, w3_ref[...], preferred_element_type=jnp.float32)
    t = g * nout_ref[:, :1]
    for c in range(2):
        out_ref[c] = t[:, c * 128:(c + 1) * 128]


_layer23_call = pl.pallas_call(
    _layer23_body,
    grid=(GRID,),
    in_specs=[
        pl.BlockSpec((NC, 4, R, 128), lambda i: (0, 0, i, 0)),
        pl.BlockSpec((R, 16), lambda i: (i, 0)),
        pl.BlockSpec((R, 16), lambda i: (i, 0)),
        pl.BlockSpec((512, 512), lambda i: (0, 0)),
        pl.BlockSpec((512, 256), lambda i: (0, 0)),
    ],
    out_specs=pl.BlockSpec((2, R, 128), lambda i: (0, i, 0)),
    out_shape=jax.ShapeDtypeStruct((2, NPAD, 128), jnp.float32),
)


def _final_body(r_ref, nin_ref, out_ref):
    out_ref[...] = jnp.concatenate(
        [r_ref[0, c] + r_ref[1, c] for c in range(2)], axis=1
    ) * nin_ref[:, :1]


_final_call = pl.pallas_call(
    _final_body,
    grid=(GRID,),
    in_specs=[
        pl.BlockSpec((NC, 2, R, 128), lambda i: (0, 0, i, 0)),
        pl.BlockSpec((R, 16), lambda i: (i, 0)),
    ],
    out_specs=pl.BlockSpec((R, 256), lambda i: (i, 0)),
    out_shape=jax.ShapeDtypeStruct((NPAD, 256), jnp.float32),
)


@jax.jit
def kernel(x, edge_index, W1, W2, W3):
    src = edge_index[0].astype(jnp.int32)
    dst = edge_index[1].astype(jnp.int32)
    pad = jnp.full((EPAD - E,), N, jnp.int32)
    src_p = jnp.concatenate([src, pad]).reshape(NW, NBLK, K)
    dst_p = jnp.concatenate([dst, pad]).reshape(NW, NBLK, K)
    x_pad = jnp.pad(x, ((0, NPAD - N), (0, 0)))
    ones128 = jnp.ones((K, 128), jnp.float32)
    zeros128 = jnp.zeros((RPT, 128), jnp.float32)

    deg = _deg_call(src_p, dst_p, ones128, zeros128)
    nout16, nin16, c1 = _prep_call(deg, x_pad)
    r1 = _agg1(c1, src_p, dst_p, zeros128)
    c2 = _layer1_call(r1, nin16, nout16, W1)
    r2 = _agg4(c2[0], c2[1], c2[2], c2[3], src_p, dst_p, zeros128)
    c3 = _layer23_call(r2, nin16, nout16, W2, W3)
    r3 = _agg2(c3[0], c3[1], src_p, dst_p, zeros128)
    out = _final_call(r3, nin16)
    return out[:N]
